# baseline (speedup 1.0000x reference)
import functools

import jax
import jax.numpy as jnp
from jax import lax
from jax.experimental import pallas as pl
from jax.experimental.pallas import tpu as pltpu
from jax.experimental.pallas import tpu_sc as plsc

EMBED = 32
SEQ = 200
LANES = 16
IDX_MINOR = 100
ROWS_PER_CHUNK = 800
IDX_ROWS = ROWS_PER_CHUNK // IDX_MINOR


def _embed_kernel(rows_per_worker, x_hbm, tok_hbm, pos_hbm, out_hbm,
                  idx_v, rows_v, pos_v, gat_sem, out_sem):
    wid = lax.axis_index("s") * 2 + lax.axis_index("c")
    base_row = wid * rows_per_worker
    n_chunks = rows_per_worker // ROWS_PER_CHUNK

    pltpu.sync_copy(pos_hbm, pos_v)

    def stage_chunk(c, slot):
        row0 = pl.multiple_of(base_row + c * ROWS_PER_CHUNK, ROWS_PER_CHUNK)
        idx_row0 = pl.multiple_of(row0 // IDX_MINOR, 8)
        pltpu.sync_copy(x_hbm.at[pl.ds(idx_row0, IDX_ROWS)], idx_v.at[slot])
        for g in range(IDX_ROWS):
            pltpu.make_async_copy(
                tok_hbm.at[idx_v.at[slot, g]],
                rows_v.at[slot, pl.ds(g * IDX_MINOR, IDX_MINOR)],
                gat_sem.at[slot]).start()

    def wait_gathers(slot):
        pltpu.make_async_copy(
            out_hbm.at[pl.ds(0, ROWS_PER_CHUNK)], rows_v.at[slot],
            gat_sem.at[slot]).wait()

    def out_copy(c, slot):
        row0 = pl.multiple_of(base_row + c * ROWS_PER_CHUNK, ROWS_PER_CHUNK)
        return pltpu.make_async_copy(
            rows_v.at[slot], out_hbm.at[pl.ds(row0, ROWS_PER_CHUNK)],
            out_sem.at[slot])

    def add_pos(slot):
        def add_body(i, carry):
            p0 = pos_v[i, pl.ds(0, LANES)]
            p1 = pos_v[i, pl.ds(LANES, LANES)]
            for s in range(ROWS_PER_CHUNK // SEQ):
                r = s * SEQ + i
                rows_v[slot, r, pl.ds(0, LANES)] = (
                    rows_v[slot, r, pl.ds(0, LANES)] + p0)
                rows_v[slot, r, pl.ds(LANES, LANES)] = (
                    rows_v[slot, r, pl.ds(LANES, LANES)] + p1)
            return carry

        lax.fori_loop(0, SEQ, add_body, 0, unroll=2)

    stage_chunk(0, 0)

    def chunk_body(c, carry):
        slot = lax.rem(c, 2)
        nslot = 1 - slot

        @pl.when(c + 1 < n_chunks)
        def _():
            @pl.when(c >= 1)
            def _():
                out_copy(c - 1, nslot).wait()
            stage_chunk(c + 1, nslot)

        wait_gathers(slot)
        add_pos(slot)
        out_copy(c, slot).start()
        return carry

    lax.fori_loop(0, n_chunks, chunk_body, 0)

    out_copy(n_chunks - 2, 0).wait()
    out_copy(n_chunks - 1, 1).wait()


def kernel(x, token_table, pos_table):
    batch, seq = x.shape
    n_rows = batch * seq
    num_workers = 32
    rows_per_worker = n_rows // num_workers
    x_flat = x.reshape(n_rows // IDX_MINOR, IDX_MINOR).astype(jnp.int32)

    mesh = plsc.VectorSubcoreMesh(core_axis_name="c", subcore_axis_name="s")
    run = pl.kernel(
        functools.partial(_embed_kernel, rows_per_worker),
        mesh=mesh,
        out_type=jax.ShapeDtypeStruct((n_rows, EMBED), jnp.float32),
        scratch_types=[
            pltpu.VMEM((2, IDX_ROWS, IDX_MINOR), jnp.int32),
            pltpu.VMEM((2, ROWS_PER_CHUNK, EMBED), jnp.float32),
            pltpu.VMEM((SEQ, EMBED), jnp.float32),
            pltpu.SemaphoreType.DMA((2,)),
            pltpu.SemaphoreType.DMA((2,)),
        ],
        compiler_params=pltpu.CompilerParams(use_tc_tiling_on_sc=False),
    )
    out = run(x_flat, token_table, pos_table)
    return out.reshape(batch, seq, EMBED)
